# Initial kernel scaffold; baseline (speedup 1.0000x reference)
import functools

import jax
import jax.numpy as jnp
from jax import lax
from jax.experimental import pallas as pl
from jax.experimental.pallas import tpu as pltpu
from jax.experimental.pallas import tpu_sc as plsc

L = 50
D = 64
NSEG = 24576
NC = 2
NS = 16
NW = NC * NS
SEG_PER_W = NSEG // NW
K = 8
NB = SEG_PER_W // K


def _pool_body(ids_hbm, table_hbm, out_hbm, idx_v, rows_v, out_v, gsem):
    c = lax.axis_index("c")
    s = lax.axis_index("s")
    wid = s * NC + c
    seg0 = wid * SEG_PER_W

    def issue(block, slot):
        base = seg0 + block * K
        pltpu.sync_copy(ids_hbm.at[pl.ds(base, K)],
                        idx_v.at[pl.ds(slot * K, K)])
        for k in range(K):
            pltpu.async_copy(table_hbm.at[idx_v.at[slot * K + k]],
                             rows_v.at[pl.ds((slot * K + k) * L, L)],
                             gsem.at[slot])

    issue(0, 0)

    def step(g, carry):
        slot = lax.rem(g, 2)

        @pl.when(g + 1 < NB)
        def _():
            issue(g + 1, 1 - slot)

        for k in range(K):
            pltpu.make_async_copy(table_hbm.at[idx_v.at[slot * K + k]],
                                  rows_v.at[pl.ds((slot * K + k) * L, L)],
                                  gsem.at[slot]).wait()

        def seg_body(k, carry2):
            srow = (slot * K + k) * L
            accs = [rows_v[srow, pl.ds(d * 16, 16)] for d in range(4)]
            for r in range(1, L):
                for d in range(4):
                    accs[d] = accs[d] + rows_v[srow + r, pl.ds(d * 16, 16)]
            m = [a * jnp.float32(1.0 / L) for a in accs]
            ssv = m[0] * m[0] + m[1] * m[1] + m[2] * m[2] + m[3] * m[3]
            ss = jnp.sum(ssv)
            sv = jnp.full((16,), ss, jnp.float32)
            ii = lax.bitcast_convert_type(sv, jnp.int32)
            yi = jnp.int32(0x5F3759DF) - lax.shift_right_arithmetic(ii, 1)
            y = lax.bitcast_convert_type(yi, jnp.float32)
            for _ in range(3):
                y = y * (jnp.float32(1.5) - jnp.float32(0.5) * sv * y * y)
            y = jnp.minimum(y, jnp.float32(1e12))
            for d in range(4):
                out_v[slot * K + k, pl.ds(d * 16, 16)] = m[d] * y
            return carry2

        lax.fori_loop(0, K, seg_body, 0)
        pltpu.sync_copy(out_v.at[pl.ds(slot * K, K)],
                        out_hbm.at[pl.ds(seg0 + g * K, K)])
        return carry

    lax.fori_loop(0, NB, step, 0)


@jax.jit
def _pooled_normalized(ids, table):
    run = pl.kernel(
        _pool_body,
        out_type=jax.ShapeDtypeStruct((NSEG, D), jnp.float32),
        mesh=plsc.VectorSubcoreMesh(core_axis_name="c", subcore_axis_name="s",
                                    num_cores=NC, num_subcores=NS),
        scratch_types=[
            pltpu.VMEM((2 * K, L), jnp.int32),
            pltpu.VMEM((2 * K * L, D), jnp.float32),
            pltpu.VMEM((2 * K, D), jnp.float32),
            pltpu.SemaphoreType.DMA((2,)),
        ],
    )
    return run(ids, table)


def kernel(anchor_input_ids, positive_input_ids, negative_input_ids,
           embedding_weight):
    ids = jnp.concatenate(
        [
            anchor_input_ids.reshape(-1, L).astype(jnp.int32),
            positive_input_ids.reshape(-1, L).astype(jnp.int32),
            negative_input_ids.reshape(-1, L).astype(jnp.int32),
        ],
        axis=0,
    )
    out = _pooled_normalized(ids, embedding_weight)
    n_a = anchor_input_ids.shape[0]
    n_p = positive_input_ids.shape[0]
    return out[:n_a], out[n_a:n_a + n_p], out[n_a + n_p:]


# speedup vs baseline: 3.0124x; 3.0124x over previous
import functools

import jax
import jax.numpy as jnp
from jax import lax
from jax.experimental import pallas as pl
from jax.experimental.pallas import tpu as pltpu
from jax.experimental.pallas import tpu_sc as plsc

L = 50
D = 64
NSEG = 24576
NC = 2
NS = 16
NW = NC * NS
SEG_PER_W = NSEG // NW
K = 8
NB = SEG_PER_W // K


def _pool_body(ids_hbm, table_hbm, out_hbm, idx_v, rows_v, out_v, gsem):
    c = lax.axis_index("c")
    s = lax.axis_index("s")
    wid = s * NC + c
    seg0 = wid * SEG_PER_W

    def issue(block, slot):
        base = seg0 + block * K
        pltpu.sync_copy(ids_hbm.at[pl.ds(base, K)],
                        idx_v.at[pl.ds(slot * K, K)])
        for k in range(K):
            pltpu.async_copy(table_hbm.at[idx_v.at[slot * K + k]],
                             rows_v.at[pl.ds((slot * K + k) * L, L)],
                             gsem.at[slot])

    issue(0, 0)

    def step(g, carry):
        slot = lax.rem(g, 2)

        @pl.when(g + 1 < NB)
        def _():
            issue(g + 1, 1 - slot)

        for k in range(K):
            pltpu.make_async_copy(table_hbm.at[idx_v.at[slot * K + k]],
                                  rows_v.at[pl.ds((slot * K + k) * L, L)],
                                  gsem.at[slot]).wait()

        def seg_body(k, carry2):
            srow = (slot * K + k) * L
            accs = [rows_v[srow, pl.ds(d * 16, 16)] for d in range(4)]
            for r in range(1, L):
                for d in range(4):
                    accs[d] = accs[d] + rows_v[srow + r, pl.ds(d * 16, 16)]
            m = [a * jnp.float32(1.0 / L) for a in accs]
            ssv = m[0] * m[0] + m[1] * m[1] + m[2] * m[2] + m[3] * m[3]
            lane = lax.iota(jnp.int32, 16)
            dn = lax.GatherDimensionNumbers(offset_dims=(),
                                            collapsed_slice_dims=(0,),
                                            start_index_map=(0,))
            sv = ssv
            for sh in (8, 4, 2, 1):
                perm = (lane ^ sh)[:, None]
                sv = sv + lax.gather(
                    sv, perm, dn, slice_sizes=(1,),
                    mode=lax.GatherScatterMode.PROMISE_IN_BOUNDS)
            ii = lax.bitcast_convert_type(sv, jnp.int32)
            yi = jnp.int32(0x5F3759DF) - lax.shift_right_arithmetic(ii, 1)
            y = lax.bitcast_convert_type(yi, jnp.float32)
            for _ in range(3):
                y = y * (jnp.float32(1.5) - jnp.float32(0.5) * sv * y * y)
            y = jnp.minimum(y, jnp.float32(1e12))
            for d in range(4):
                out_v[slot * K + k, pl.ds(d * 16, 16)] = m[d] * y
            return carry2

        lax.fori_loop(0, K, seg_body, 0)
        pltpu.sync_copy(out_v.at[pl.ds(slot * K, K)],
                        out_hbm.at[pl.ds(seg0 + g * K, K)])
        return carry

    lax.fori_loop(0, NB, step, 0)


@jax.jit
def _pooled_normalized(ids, table):
    run = pl.kernel(
        _pool_body,
        out_type=jax.ShapeDtypeStruct((NSEG, D), jnp.float32),
        mesh=plsc.VectorSubcoreMesh(core_axis_name="c", subcore_axis_name="s",
                                    num_cores=NC, num_subcores=NS),
        scratch_types=[
            pltpu.VMEM((2 * K, L), jnp.int32),
            pltpu.VMEM((2 * K * L, D), jnp.float32),
            pltpu.VMEM((2 * K, D), jnp.float32),
            pltpu.SemaphoreType.DMA((2,)),
        ],
        compiler_params=pltpu.CompilerParams(use_tc_tiling_on_sc=False),
    )
    return run(ids, table)


def kernel(anchor_input_ids, positive_input_ids, negative_input_ids,
           embedding_weight):
    ids = jnp.concatenate(
        [
            anchor_input_ids.reshape(-1, L).astype(jnp.int32),
            positive_input_ids.reshape(-1, L).astype(jnp.int32),
            negative_input_ids.reshape(-1, L).astype(jnp.int32),
        ],
        axis=0,
    )
    out = _pooled_normalized(ids, embedding_weight)
    n_a = anchor_input_ids.shape[0]
    n_p = positive_input_ids.shape[0]
    return out[:n_a], out[n_a:n_a + n_p], out[n_a + n_p:]


# speedup vs baseline: 3.0363x; 1.0079x over previous
import jax
import jax.numpy as jnp
from jax import lax
from jax.experimental import pallas as pl
from jax.experimental.pallas import tpu as pltpu
from jax.experimental.pallas import tpu_sc as plsc

L = 50
D = 64
NC = 2
NS = 16
NW = NC * NS
K = 8

B_A = 4096
B_N = 16384


def _pool_body(a_ids, p_ids, n_ids, table_hbm, a_out, p_out, n_out,
               idx_v, rows_v, out_v, gsem):
    c = lax.axis_index("c")
    s = lax.axis_index("s")
    wid = s * NC + c

    def process(ids_hbm, out_hbm, seg_per_w):
        nb = seg_per_w // K
        seg0 = wid * seg_per_w

        def issue(block, slot):
            base = seg0 + block * K
            pltpu.sync_copy(ids_hbm.at[pl.ds(base, K)],
                            idx_v.at[pl.ds(slot * K, K)])
            for k in range(K):
                pltpu.async_copy(table_hbm.at[idx_v.at[slot * K + k]],
                                 rows_v.at[pl.ds((slot * K + k) * L, L)],
                                 gsem.at[slot])

        issue(0, 0)

        def step(g, carry):
            slot = lax.rem(g, 2)

            @pl.when(g + 1 < nb)
            def _():
                issue(g + 1, 1 - slot)

            for k in range(K):
                pltpu.make_async_copy(
                    table_hbm.at[idx_v.at[slot * K + k]],
                    rows_v.at[pl.ds((slot * K + k) * L, L)],
                    gsem.at[slot]).wait()

            def seg_body(k, carry2):
                srow = (slot * K + k) * L
                accs = [rows_v[srow, pl.ds(d * 16, 16)] for d in range(4)]
                for r in range(1, L):
                    for d in range(4):
                        accs[d] = accs[d] + rows_v[srow + r, pl.ds(d * 16, 16)]
                m = [a * jnp.float32(1.0 / L) for a in accs]
                ssv = m[0] * m[0] + m[1] * m[1] + m[2] * m[2] + m[3] * m[3]
                lane = lax.iota(jnp.int32, 16)
                dn = lax.GatherDimensionNumbers(offset_dims=(),
                                                collapsed_slice_dims=(0,),
                                                start_index_map=(0,))
                sv = ssv
                for sh in (8, 4, 2, 1):
                    perm = (lane ^ sh)[:, None]
                    sv = sv + lax.gather(
                        sv, perm, dn, slice_sizes=(1,),
                        mode=lax.GatherScatterMode.PROMISE_IN_BOUNDS)
                ii = lax.bitcast_convert_type(sv, jnp.int32)
                yi = jnp.int32(0x5F3759DF) - lax.shift_right_arithmetic(ii, 1)
                y = lax.bitcast_convert_type(yi, jnp.float32)
                for _ in range(3):
                    y = y * (jnp.float32(1.5) - jnp.float32(0.5) * sv * y * y)
                y = jnp.minimum(y, jnp.float32(1e12))
                for d in range(4):
                    out_v[slot * K + k, pl.ds(d * 16, 16)] = m[d] * y
                return carry2

            lax.fori_loop(0, K, seg_body, 0)
            pltpu.sync_copy(out_v.at[pl.ds(slot * K, K)],
                            out_hbm.at[pl.ds(seg0 + g * K, K)])
            return carry

        lax.fori_loop(0, nb, step, 0)

    process(a_ids, a_out, B_A // NW)
    process(p_ids, p_out, B_A // NW)
    process(n_ids, n_out, B_N // NW)


@jax.jit
def _pooled_normalized(a_ids, p_ids, n_ids, table):
    run = pl.kernel(
        _pool_body,
        out_type=(
            jax.ShapeDtypeStruct((B_A, D), jnp.float32),
            jax.ShapeDtypeStruct((B_A, D), jnp.float32),
            jax.ShapeDtypeStruct((B_N, D), jnp.float32),
        ),
        mesh=plsc.VectorSubcoreMesh(core_axis_name="c", subcore_axis_name="s",
                                    num_cores=NC, num_subcores=NS),
        scratch_types=[
            pltpu.VMEM((2 * K, L), jnp.int32),
            pltpu.VMEM((2 * K * L, D), jnp.float32),
            pltpu.VMEM((2 * K, D), jnp.float32),
            pltpu.SemaphoreType.DMA((2,)),
        ],
        compiler_params=pltpu.CompilerParams(use_tc_tiling_on_sc=False),
    )
    return run(a_ids, p_ids, n_ids, table)


def kernel(anchor_input_ids, positive_input_ids, negative_input_ids,
           embedding_weight):
    return _pooled_normalized(
        anchor_input_ids.astype(jnp.int32),
        positive_input_ids.astype(jnp.int32),
        negative_input_ids.reshape(-1, L).astype(jnp.int32),
        embedding_weight,
    )
